# Initial kernel scaffold; baseline (speedup 1.0000x reference)
import functools

import jax
import jax.numpy as jnp
from jax import lax
from jax.experimental import pallas as pl
from jax.experimental.pallas import tpu as pltpu
from jax.experimental.pallas import tpu_sc as plsc

VOCAB = 1000000
DIM = 64
BATCH = 16384
HIST = 50

_INFO = plsc.get_sparse_core_info()
NC = _INFO.num_cores
NS = _INFO.num_subcores
NW = NC * NS

TOTAL = BATCH * HIST
CHUNK = 128
ROWS_PER_W = TOTAL // NW
CHUNKS_PER_W = ROWS_PER_W // CHUNK

_mesh = plsc.VectorSubcoreMesh(core_axis_name="c", subcore_axis_name="s")


@functools.partial(
    pl.kernel,
    out_type=jax.ShapeDtypeStruct((TOTAL, DIM), jnp.float32),
    mesh=_mesh,
    scratch_types=[
        pltpu.VMEM((CHUNKS_PER_W, CHUNK), jnp.int32),
        pltpu.VMEM((CHUNK, DIM), jnp.float32),
        pltpu.SemaphoreType.DMA,
    ],
)
def _gather_kernel(idx_hbm, table_hbm, out_hbm, idx_v, rows_v, sem):
    wid = lax.axis_index("s") * NC + lax.axis_index("c")
    row_base = wid * ROWS_PER_W
    pltpu.sync_copy(idx_hbm.at[pl.ds(wid * CHUNKS_PER_W, CHUNKS_PER_W)], idx_v)

    @pl.loop(0, CHUNKS_PER_W)
    def _chunk(c):
        pltpu.async_copy(table_hbm.at[idx_v.at[c]], rows_v, sem).wait()
        pltpu.sync_copy(rows_v, out_hbm.at[pl.ds(row_base + c * CHUNK, CHUNK)])


def kernel(x, table):
    idx = x.reshape(TOTAL // CHUNK, CHUNK).astype(jnp.int32)
    out = _gather_kernel(idx, table)
    return out.reshape(BATCH, HIST, DIM)


# speedup vs baseline: 1.8713x; 1.8713x over previous
import functools

import jax
import jax.numpy as jnp
from jax import lax
from jax.experimental import pallas as pl
from jax.experimental.pallas import tpu as pltpu
from jax.experimental.pallas import tpu_sc as plsc

VOCAB = 1000000
DIM = 64
BATCH = 16384
HIST = 50

_INFO = plsc.get_sparse_core_info()
NC = _INFO.num_cores
NS = _INFO.num_subcores
NW = NC * NS

TOTAL = BATCH * HIST
CHUNK = 128
GPB = 4
BIG = CHUNK * GPB
ROWS_PER_W = TOTAL // NW
CHUNKS_PER_W = ROWS_PER_W // CHUNK
ROUNDS = CHUNKS_PER_W // GPB

_mesh = plsc.VectorSubcoreMesh(core_axis_name="c", subcore_axis_name="s")


@functools.partial(
    pl.kernel,
    out_type=jax.ShapeDtypeStruct((TOTAL, DIM), jnp.float32),
    mesh=_mesh,
    scratch_types=[
        pltpu.VMEM((CHUNKS_PER_W, CHUNK), jnp.int32),
        pltpu.VMEM((BIG, DIM), jnp.float32),
        pltpu.VMEM((BIG, DIM), jnp.float32),
        pltpu.SemaphoreType.DMA,
        pltpu.SemaphoreType.DMA,
        pltpu.SemaphoreType.DMA,
        pltpu.SemaphoreType.DMA,
    ],
    compiler_params=pltpu.CompilerParams(use_tc_tiling_on_sc=False),
)
def _gather_kernel(idx_hbm, table_hbm, out_hbm, idx_v, big0, big1,
                   g0, g1, s0, s1):
    wid = lax.axis_index("s") * NC + lax.axis_index("c")
    row_base = wid * ROWS_PER_W
    bigs = (big0, big1)
    gsems = (g0, g1)
    ssems = (s0, s1)

    pltpu.sync_copy(idx_hbm.at[pl.ds(wid * CHUNKS_PER_W, CHUNKS_PER_W)], idx_v)

    def fire_gathers(r, buf, sem):
        for j in range(GPB):
            pltpu.async_copy(
                table_hbm.at[idx_v.at[r * GPB + j]],
                buf.at[pl.ds(j * CHUNK, CHUNK)],
                sem,
            )

    def drain_gathers(r, buf, sem):
        for j in range(GPB):
            pltpu.make_async_copy(
                table_hbm.at[idx_v.at[r * GPB + j]],
                buf.at[pl.ds(j * CHUNK, CHUNK)],
                sem,
            ).wait()

    def store(r, buf, sem):
        return pltpu.async_copy(
            buf, out_hbm.at[pl.ds(row_base + r * BIG, BIG)], sem)

    def wait_store(r, buf, sem):
        pltpu.make_async_copy(
            buf, out_hbm.at[pl.ds(row_base + r * BIG, BIG)], sem).wait()

    @pl.loop(0, ROUNDS, step=2)
    def _round(r):
        for b in range(2):
            rb = r + b

            @pl.when(rb >= 2)
            def _():
                wait_store(rb - 2, bigs[b], ssems[b])

            fire_gathers(rb, bigs[b], gsems[b])
            drain_gathers(rb, bigs[b], gsems[b])
            store(rb, bigs[b], ssems[b])

    for b in range(2):
        wait_store(ROUNDS - 2 + b, bigs[b], ssems[b])


def kernel(x, table):
    idx = x.reshape(TOTAL // CHUNK, CHUNK).astype(jnp.int32)
    out = _gather_kernel(idx, table)
    return out.reshape(BATCH, HIST, DIM)


# speedup vs baseline: 1.8728x; 1.0008x over previous
import functools

import jax
import jax.numpy as jnp
from jax import lax
from jax.experimental import pallas as pl
from jax.experimental.pallas import tpu as pltpu
from jax.experimental.pallas import tpu_sc as plsc

VOCAB = 1000000
DIM = 64
BATCH = 16384
HIST = 50

_INFO = plsc.get_sparse_core_info()
NC = _INFO.num_cores
NS = _INFO.num_subcores
NW = NC * NS

TOTAL = BATCH * HIST
CHUNK = 128
GPB = 4
BIG = CHUNK * GPB
ROWS_PER_W = TOTAL // NW
CHUNKS_PER_W = ROWS_PER_W // CHUNK
ROUNDS = CHUNKS_PER_W // GPB

_mesh = plsc.VectorSubcoreMesh(core_axis_name="c", subcore_axis_name="s")


@functools.partial(
    pl.kernel,
    out_type=jax.ShapeDtypeStruct((TOTAL, DIM), jnp.float32),
    mesh=_mesh,
    scratch_types=[
        pltpu.VMEM((ROWS_PER_W,), jnp.int32),
        pltpu.VMEM((BIG, DIM), jnp.float32),
        pltpu.VMEM((BIG, DIM), jnp.float32),
        pltpu.SemaphoreType.DMA,
        pltpu.SemaphoreType.DMA,
        pltpu.SemaphoreType.DMA,
        pltpu.SemaphoreType.DMA,
    ],
    compiler_params=pltpu.CompilerParams(use_tc_tiling_on_sc=False),
)
def _gather_kernel(idx_hbm, table_hbm, out_hbm, idx_v, big0, big1,
                   g0, g1, s0, s1):
    wid = lax.axis_index("s") * NC + lax.axis_index("c")
    row_base = wid * ROWS_PER_W
    bigs = (big0, big1)
    gsems = (g0, g1)
    ssems = (s0, s1)

    pltpu.sync_copy(idx_hbm.at[pl.ds(row_base, ROWS_PER_W)], idx_v)

    def fire_gathers(r, buf, sem):
        for j in range(GPB):
            pltpu.async_copy(
                table_hbm.at[idx_v.at[pl.ds((r * GPB + j) * CHUNK, CHUNK)]],
                buf.at[pl.ds(j * CHUNK, CHUNK)],
                sem,
            )

    def drain_gathers(r, buf, sem):
        for j in range(GPB):
            pltpu.make_async_copy(
                table_hbm.at[idx_v.at[pl.ds((r * GPB + j) * CHUNK, CHUNK)]],
                buf.at[pl.ds(j * CHUNK, CHUNK)],
                sem,
            ).wait()

    def store(r, buf, sem):
        return pltpu.async_copy(
            buf, out_hbm.at[pl.ds(row_base + r * BIG, BIG)], sem)

    def wait_store(r, buf, sem):
        pltpu.make_async_copy(
            buf, out_hbm.at[pl.ds(row_base + r * BIG, BIG)], sem).wait()

    @pl.loop(0, ROUNDS, step=2)
    def _round(r):
        for b in range(2):
            rb = r + b

            @pl.when(rb >= 2)
            def _():
                wait_store(rb - 2, bigs[b], ssems[b])

            fire_gathers(rb, bigs[b], gsems[b])
            drain_gathers(rb, bigs[b], gsems[b])
            store(rb, bigs[b], ssems[b])

    for b in range(2):
        wait_store(ROUNDS - 2 + b, bigs[b], ssems[b])


def kernel(x, table):
    idx = x.reshape(TOTAL).astype(jnp.int32)
    out = _gather_kernel(idx, table)
    return out.reshape(BATCH, HIST, DIM)
